# baseline (speedup 1.0000x reference)
import jax
import jax.numpy as jnp
from jax import lax
from jax.experimental import pallas as pl
from jax.experimental.pallas import tpu as pltpu
from jax.experimental.pallas import tpu_sc as plsc

H = 32
W = 32
D = 256
B = 32
F = 2 * D
LANES = 16


def _pos_body(row_hbm, col_hbm, pos_hbm, rowv, build_v):
    c = lax.axis_index("c")
    s = lax.axis_index("s")
    wid = c * 16 + s

    pltpu.sync_copy(col_hbm, build_v.at[:, pl.ds(0, D)])

    pltpu.sync_copy(row_hbm.at[pl.ds(wid, 1)], rowv)

    vs = [rowv[0, pl.ds(j * LANES, LANES)] for j in range(D // LANES)]

    def st(i, carry):
        for j in range(D // LANES):
            build_v[i, pl.ds(D + j * LANES, LANES)] = vs[j]
        return carry

    lax.fori_loop(0, W, st, 0)

    pltpu.sync_copy(build_v, pos_hbm.at[pl.ds(wid * W, W)])


_pos_sc = pl.kernel(
    _pos_body,
    out_type=jax.ShapeDtypeStruct((H * W, F), jnp.float32),
    mesh=plsc.VectorSubcoreMesh(core_axis_name="c", subcore_axis_name="s"),
    scratch_types=[
        pltpu.VMEM((1, D), jnp.float32),
        pltpu.VMEM((W, F), jnp.float32),
    ],
)


def _bc_body(pos_ref, o_ref):
    o_ref[...] = pos_ref[...][None]


_bc_tc = pl.pallas_call(
    _bc_body,
    out_shape=jax.ShapeDtypeStruct((B, H * W, F), jnp.float32),
    grid=(B,),
    in_specs=[pl.BlockSpec((H * W, F), lambda b: (0, 0))],
    out_specs=pl.BlockSpec((1, H * W, F), lambda b: (b, 0, 0)),
)


def kernel(x, row_w, col_w):
    del x
    pos = _pos_sc(row_w, col_w)
    return _bc_tc(pos)


# speedup vs baseline: 1.0548x; 1.0548x over previous
import jax
import jax.numpy as jnp
from jax import lax
from jax.experimental import pallas as pl
from jax.experimental.pallas import tpu as pltpu
from jax.experimental.pallas import tpu_sc as plsc

H = 32
W = 32
D = 256
B = 32
F = 2 * D
LANES = 16


def _pos_body(row_hbm, col_hbm, pos_hbm, rowv, build_v):
    c = lax.axis_index("c")
    s = lax.axis_index("s")
    wid = c * 16 + s

    pltpu.sync_copy(col_hbm, build_v.at[:, pl.ds(0, D)])

    pltpu.sync_copy(row_hbm.at[pl.ds(wid, 1)], rowv)

    vs = [rowv[0, pl.ds(j * LANES, LANES)] for j in range(D // LANES)]

    def st(i, carry):
        for j in range(D // LANES):
            build_v[i, pl.ds(D + j * LANES, LANES)] = vs[j]
        return carry

    lax.fori_loop(0, W, st, 0)

    pltpu.sync_copy(build_v, pos_hbm.at[pl.ds(wid * W, W)])


_pos_sc = pl.kernel(
    _pos_body,
    out_type=jax.ShapeDtypeStruct((H * W, F), jnp.float32),
    mesh=plsc.VectorSubcoreMesh(core_axis_name="c", subcore_axis_name="s"),
    scratch_types=[
        pltpu.VMEM((1, D), jnp.float32),
        pltpu.VMEM((W, F), jnp.float32),
    ],
)


def _bc_body(pos_hbm, out_hbm, pos_v, sem_in, sem_out):
    pltpu.make_async_copy(pos_hbm, pos_v, sem_in).start()
    pltpu.make_async_copy(pos_hbm, pos_v, sem_in).wait()
    copies = [
        pltpu.make_async_copy(pos_v, out_hbm.at[b], sem_out) for b in range(B)
    ]
    for cp in copies:
        cp.start()
    for cp in copies:
        cp.wait()


_bc_tc = pl.pallas_call(
    _bc_body,
    out_shape=jax.ShapeDtypeStruct((B, H * W, F), jnp.float32),
    in_specs=[pl.BlockSpec(memory_space=pl.ANY)],
    out_specs=pl.BlockSpec(memory_space=pl.ANY),
    scratch_shapes=[
        pltpu.VMEM((H * W, F), jnp.float32),
        pltpu.SemaphoreType.DMA,
        pltpu.SemaphoreType.DMA,
    ],
)


def kernel(x, row_w, col_w):
    del x
    pos = _pos_sc(row_w, col_w)
    return _bc_tc(pos)
